# baseline (speedup 1.0000x reference)
import functools

import jax
import jax.numpy as jnp
from jax import lax
from jax.experimental import pallas as pl
from jax.experimental.pallas import tpu as pltpu
from jax.experimental.pallas import tpu_sc as plsc

_F = 26
_D = 32
_B = 16384
_OFFSETS = (1, 5, 7, 11, 13)
_E = _F * len(_OFFSETS)

_NC = 2
_NS = 16
_NW = _NC * _NS
_RPW = _B // _NW
_NB = 128
_NCHUNK = _RPW // _NB
_GRP = _D * (_NB // 16)


def _edge_table():
    pairs = sorted(((f + o) % _F, f) for f in range(_F) for o in _OFFSETS)
    eid = {p: k for k, p in enumerate(pairs)}
    return tuple(
        tuple(((f + o) % _F, eid[((f + o) % _F, f)]) for o in _OFFSETS)
        for f in range(_F)
    )


_TABLE = _edge_table()


def _tanh(x):
    return 1.0 - 2.0 / (jnp.exp(x * 2.0) + 1.0)


def _mac(hs, wv, sl, layer, f):
    t = [hs[s] * wv[sl, pl.ds((layer * _E + e) * 16, 16)]
         for s, e in _TABLE[f]]
    return ((t[0] + t[1]) + (t[2] + t[3])) + t[4]


@functools.partial(
    pl.kernel,
    mesh=plsc.VectorSubcoreMesh(core_axis_name="c", subcore_axis_name="s"),
    out_type=jax.ShapeDtypeStruct((_F * _D, _B), jnp.float32),
    scratch_types=[
        pltpu.VMEM((_F * _D, _NB), jnp.float32),
        pltpu.VMEM((_F, 16), jnp.float32),
        pltpu.VMEM((2, 2 * _E * 16), jnp.float32),
        pltpu.SMEM((_E,), jnp.float32),
        pltpu.SMEM((_E,), jnp.float32),
        pltpu.VMEM_SHARED((_E,), jnp.float32),
        pltpu.VMEM_SHARED((_E,), jnp.float32),
        pltpu.VMEM_SHARED((_NS, 2 * _E * 16), jnp.float32),
    ],
)
def _gnn(x_hbm, w0_hbm, w1_hbm, out_hbm,
         buf_v, h1_v, wv, w0_s, w1_s, w0_vs, w1_vs, wsp_vs):
    wid = lax.axis_index("s") * _NC + lax.axis_index("c")
    sid = lax.axis_index("s")
    base = wid * _RPW
    pltpu.sync_copy(w0_hbm, w0_vs)
    pltpu.sync_copy(w1_hbm, w1_vs)
    pltpu.sync_copy(w0_vs, w0_s)
    pltpu.sync_copy(w1_vs, w1_s)

    for layer, w_s in enumerate((w0_s, w1_s)):
        def wsetup(e, c, w_s=w_s, layer=layer):
            wv[0, pl.ds((layer * _E + e) * 16, 16)] = jnp.full(
                (16,), w_s[e], jnp.float32)
            return c

        lax.fori_loop(0, _E, wsetup, 0)
    pltpu.sync_copy(wv.at[0], wsp_vs.at[sid])
    pltpu.sync_copy(wsp_vs.at[sid], wv.at[1])

    def chunk(ci, carry):
        b0 = base + ci * _NB
        pltpu.sync_copy(x_hbm.at[:, pl.ds(b0, _NB)], buf_v)

        def colfn(gi, c2):
            d = gi // (_NB // 16)
            lb = (gi % (_NB // 16)) * 16
            sl = jnp.bitwise_and(gi, 1)
            hs = [buf_v[f * _D + d, pl.ds(lb, 16)] for f in range(_F)]
            for f in range(_F):
                h1_v[f, :] = _tanh(_mac(hs, wv, sl, 0, f))
            hs = [h1_v[f, :] for f in range(_F)]
            for f in range(_F):
                buf_v[f * _D + d, pl.ds(lb, 16)] = _tanh(_mac(hs, wv, sl, 1, f))
            return c2

        lax.fori_loop(0, _GRP, colfn, 0)
        pltpu.sync_copy(buf_v, out_hbm.at[:, pl.ds(b0, _NB)])
        return carry

    lax.fori_loop(0, _NCHUNK, chunk, 0)


def kernel(inputs, w0, w1):
    xt = jnp.transpose(inputs, (1, 2, 0)).reshape(_F * _D, _B)
    out = _gnn(xt, w0, w1)
    return jnp.transpose(out, (1, 0))


# speedup vs baseline: 1.0007x; 1.0007x over previous
import functools

import jax
import jax.numpy as jnp
from jax import lax
from jax.experimental import pallas as pl
from jax.experimental.pallas import tpu as pltpu
from jax.experimental.pallas import tpu_sc as plsc

_F = 26
_D = 32
_B = 16384
_OFFSETS = (1, 5, 7, 11, 13)
_E = _F * len(_OFFSETS)

_NC = 2
_NS = 16
_NW = _NC * _NS
_RPW = _B // _NW
_NB = 128
_NCHUNK = _RPW // _NB
_GRP = _D * (_NB // 16)


def _edge_table():
    pairs = sorted(((f + o) % _F, f) for f in range(_F) for o in _OFFSETS)
    eid = {p: k for k, p in enumerate(pairs)}
    return tuple(
        tuple(((f + o) % _F, eid[((f + o) % _F, f)]) for o in _OFFSETS)
        for f in range(_F)
    )


_TABLE = _edge_table()


def _tanh(x):
    return 1.0 - 2.0 / (jnp.exp(x * 2.0) + 1.0)


def _mac(hs, wv, sl, layer, f):
    t = [hs[s] * wv[sl, pl.ds((layer * _E + e) * 16, 16)]
         for s, e in _TABLE[f]]
    return ((t[0] + t[1]) + (t[2] + t[3])) + t[4]


@functools.partial(
    pl.kernel,
    mesh=plsc.VectorSubcoreMesh(core_axis_name="c", subcore_axis_name="s"),
    out_type=jax.ShapeDtypeStruct((_F * _D, _B), jnp.float32),
    scratch_types=[
        pltpu.VMEM((_F * _D, _NB), jnp.float32),
        pltpu.VMEM((_F, 16), jnp.float32),
        pltpu.VMEM((2, 2 * _E * 16), jnp.float32),
        pltpu.SMEM((_E,), jnp.float32),
        pltpu.SMEM((_E,), jnp.float32),
        pltpu.VMEM_SHARED((_E,), jnp.float32),
        pltpu.VMEM_SHARED((_E,), jnp.float32),
        pltpu.VMEM_SHARED((_NS, 2 * _E * 16), jnp.float32),
    ],
)
def _gnn(x_hbm, w0_hbm, w1_hbm, out_hbm,
         buf_v, h1_v, wv, w0_s, w1_s, w0_vs, w1_vs, wsp_vs):
    wid = lax.axis_index("s") * _NC + lax.axis_index("c")
    sid = lax.axis_index("s")
    base = wid * _RPW
    pltpu.sync_copy(w0_hbm, w0_vs)
    pltpu.sync_copy(w1_hbm, w1_vs)
    pltpu.sync_copy(w0_vs, w0_s)
    pltpu.sync_copy(w1_vs, w1_s)

    for layer, w_s in enumerate((w0_s, w1_s)):
        for e in range(_E):
            wv[0, pl.ds((layer * _E + e) * 16, 16)] = jnp.full(
                (16,), w_s[e], jnp.float32)
    pltpu.sync_copy(wv.at[0], wsp_vs.at[sid])
    pltpu.sync_copy(wsp_vs.at[sid], wv.at[1])

    def chunk(ci, carry):
        b0 = base + ci * _NB
        pltpu.sync_copy(x_hbm.at[:, pl.ds(b0, _NB)], buf_v)

        def colfn(gi, c2):
            d = gi // (_NB // 16)
            lb = (gi % (_NB // 16)) * 16
            sl = jnp.bitwise_and(gi, 1)
            hs = [buf_v[f * _D + d, pl.ds(lb, 16)] for f in range(_F)]
            for f in range(_F):
                h1_v[f, :] = _tanh(_mac(hs, wv, sl, 0, f))
            hs = [h1_v[f, :] for f in range(_F)]
            for f in range(_F):
                buf_v[f * _D + d, pl.ds(lb, 16)] = _tanh(_mac(hs, wv, sl, 1, f))
            return c2

        lax.fori_loop(0, _GRP, colfn, 0)
        pltpu.sync_copy(buf_v, out_hbm.at[:, pl.ds(b0, _NB)])
        return carry

    lax.fori_loop(0, _NCHUNK, chunk, 0)


def kernel(inputs, w0, w1):
    xt = jnp.transpose(inputs, (1, 2, 0)).reshape(_F * _D, _B)
    out = _gnn(xt, w0, w1)
    return jnp.transpose(out, (1, 0))


# speedup vs baseline: 3.0321x; 3.0300x over previous
import functools

import jax
import jax.numpy as jnp
from jax import lax
from jax.experimental import pallas as pl
from jax.experimental.pallas import tpu as pltpu
from jax.experimental.pallas import tpu_sc as plsc

_F = 26
_D = 32
_B = 16384
_OFFSETS = (1, 5, 7, 11, 13)
_E = _F * len(_OFFSETS)

_NC = 2
_NS = 16
_NW = _NC * _NS
_RPW = _B // _NW
_NB = 128
_NCHUNK = _RPW // _NB
_GRP = _D * (_NB // 16)


def _edge_table():
    pairs = sorted(((f + o) % _F, f) for f in range(_F) for o in _OFFSETS)
    eid = {p: k for k, p in enumerate(pairs)}
    return tuple(
        tuple(((f + o) % _F, eid[((f + o) % _F, f)]) for o in _OFFSETS)
        for f in range(_F)
    )


_TABLE = _edge_table()


def _tanh(x):
    return 1.0 - 2.0 / (jnp.exp(x * 2.0) + 1.0)


def _mac(hs, w_s, f):
    t = [hs[s] * w_s[e] for s, e in _TABLE[f]]
    return ((t[0] + t[1]) + (t[2] + t[3])) + t[4]


@functools.partial(
    pl.kernel,
    mesh=plsc.VectorSubcoreMesh(core_axis_name="c", subcore_axis_name="s"),
    out_type=jax.ShapeDtypeStruct((_F * _D, _B), jnp.float32),
    scratch_types=[
        pltpu.VMEM((_F * _D, _NB), jnp.float32),
        pltpu.VMEM((_F, 16), jnp.float32),
        pltpu.SMEM((_E,), jnp.float32),
        pltpu.SMEM((_E,), jnp.float32),
        pltpu.VMEM_SHARED((_E,), jnp.float32),
        pltpu.VMEM_SHARED((_E,), jnp.float32),
    ],
)
def _gnn(x_hbm, w0_hbm, w1_hbm, out_hbm,
         buf_v, h1_v, w0_s, w1_s, w0_vs, w1_vs):
    wid = lax.axis_index("s") * _NC + lax.axis_index("c")
    base = wid * _RPW
    pltpu.sync_copy(w0_hbm, w0_vs)
    pltpu.sync_copy(w1_hbm, w1_vs)
    pltpu.sync_copy(w0_vs, w0_s)
    pltpu.sync_copy(w1_vs, w1_s)

    def chunk(ci, carry):
        b0 = base + ci * _NB
        pltpu.sync_copy(x_hbm.at[:, pl.ds(b0, _NB)], buf_v)

        def colfn(gi, c2):
            d = gi // (_NB // 16)
            lb = (gi % (_NB // 16)) * 16
            hs = [buf_v[f * _D + d, pl.ds(lb, 16)] for f in range(_F)]
            for f in range(_F):
                h1_v[f, :] = _tanh(_mac(hs, w0_s, f))
            hs = [h1_v[f, :] for f in range(_F)]
            for f in range(_F):
                buf_v[f * _D + d, pl.ds(lb, 16)] = _tanh(_mac(hs, w1_s, f))
            return c2

        lax.fori_loop(0, _GRP, colfn, 0)
        pltpu.sync_copy(buf_v, out_hbm.at[:, pl.ds(b0, _NB)])
        return carry

    lax.fori_loop(0, _NCHUNK, chunk, 0)


def kernel(inputs, w0, w1):
    xt = jnp.transpose(inputs, (1, 2, 0)).reshape(_F * _D, _B)
    out = _gnn(xt, w0, w1)
    return jnp.transpose(out, (1, 0))
